# baseline (speedup 1.0000x reference)
import functools

import jax
import jax.numpy as jnp
from jax import lax
from jax.experimental import pallas as pl
from jax.experimental.pallas import tpu as pltpu
from jax.experimental.pallas import tpu_sc as plsc

V = 1000000
D = 64
B = 16384
K = 20

NC = 2
NS = 16
NW = NC * NS
BPW = B // NW
CHUNK = 128
NCH = BPW // CHUNK
GRPS = CHUNK // 16

TBLK = 1024
NQ = (V + TBLK - 1) // TBLK
VROWS = 2 * TBLK * ((NQ + 1) // 2)


def _tr_body(x_ref, o_ref):
    q = pl.program_id(0)
    t = x_ref[...].T

    @pl.when(q % 2 == 0)
    def _():
        o_ref[:, 0:D] = t

    @pl.when(q % 2 == 1)
    def _():
        o_ref[:, D:2 * D] = t


_tc_transpose = pl.pallas_call(
    _tr_body,
    grid=(NQ,),
    in_specs=[pl.BlockSpec((D, TBLK), lambda q: (0, q))],
    out_specs=pl.BlockSpec((TBLK, 2 * D), lambda q: (q // 2, 0)),
    out_shape=jax.ShapeDtypeStruct((VROWS // 2, 2 * D), jnp.float32),
)


def _remap(r):
    q = r >> 10
    return 2048 * (q >> 1) + 2 * (r & 1023) + (q & 1)


def _dots16(buf, vbuf, scr, g):
    def row(j, _):
        r = g * 16 + j
        p = buf[r, pl.ds(0, 16)] * vbuf[r, pl.ds(0, 16)]
        p = p + buf[r, pl.ds(16, 16)] * vbuf[r, pl.ds(16, 16)]
        p = p + buf[r, pl.ds(32, 16)] * vbuf[r, pl.ds(32, 16)]
        p = p + buf[r, pl.ds(48, 16)] * vbuf[r, pl.ds(48, 16)]
        plsc.store_scatter(scr, [lax.iota(jnp.int32, 16) * 16 + j], p)
        return 0

    lax.fori_loop(0, 16, row, 0)

    def srow(i, a):
        return a + scr[pl.ds(i * 16, 16)]

    return lax.fori_loop(0, 16, srow, jnp.zeros((16,), jnp.float32))


_mesh = plsc.VectorSubcoreMesh(core_axis_name="c", subcore_axis_name="s")


@functools.partial(
    pl.kernel,
    mesh=_mesh,
    compiler_params=pltpu.CompilerParams(
        needs_layout_passes=False, use_tc_tiling_on_sc=False
    ),
    out_type=(
        jax.ShapeDtypeStruct((B,), jnp.float32),
        jax.ShapeDtypeStruct((B,), jnp.float32),
    ),
    scratch_types=[
        pltpu.VMEM((CHUNK,), jnp.int32),
        pltpu.VMEM((CHUNK,), jnp.int32),
        pltpu.VMEM((K, CHUNK), jnp.int32),
        pltpu.VMEM((CHUNK, D), jnp.float32),
        pltpu.VMEM((CHUNK, D), jnp.float32),
        pltpu.VMEM((CHUNK, D), jnp.float32),
        pltpu.VMEM((256,), jnp.float32),
        pltpu.VMEM((CHUNK,), jnp.float32),
        pltpu.VMEM((CHUNK,), jnp.float32),
        pltpu.SemaphoreType.DMA,
    ],
)
def _sc_dots(tgt, ctx, negt, iemb, oemb, pdot, ndot,
             tgti, ctxi, negi, vbuf, ubuf, nbuf, scr, pv, nv, sem):
    wid = lax.axis_index("s") * NC + lax.axis_index("c")
    for c in range(NCH):
        off = wid * BPW + c * CHUNK
        pltpu.sync_copy(tgt.at[pl.ds(off, CHUNK)], tgti)
        pltpu.sync_copy(ctx.at[pl.ds(off, CHUNK)], ctxi)
        pltpu.sync_copy(negt.at[:, pl.ds(off, CHUNK)], negi)
        pltpu.async_copy(iemb.at[tgti], vbuf, sem).wait()
        pltpu.async_copy(oemb.at[ctxi], ubuf, sem).wait()

        def pgrp(g, _):
            pv[pl.ds(g * 16, 16)] = _dots16(ubuf, vbuf, scr, g)
            return 0

        lax.fori_loop(0, GRPS, pgrp, 0)

        def zb(g, _):
            nv[pl.ds(g * 16, 16)] = jnp.zeros((16,), jnp.float32)
            return 0

        lax.fori_loop(0, GRPS, zb, 0)

        def kb(k, _):
            pltpu.async_copy(oemb.at[negi.at[k]], nbuf, sem).wait()

            def ngrp(g, _):
                nv[pl.ds(g * 16, 16)] = (
                    nv[pl.ds(g * 16, 16)] + _dots16(nbuf, vbuf, scr, g)
                )
                return 0

            lax.fori_loop(0, GRPS, ngrp, 0)
            return 0

        lax.fori_loop(0, K, kb, 0)

        pltpu.sync_copy(pv, pdot.at[pl.ds(off, CHUNK)])
        pltpu.sync_copy(nv, ndot.at[pl.ds(off, CHUNK)])


def _tc_body(p_ref, n_ref, o_ref):
    p = p_ref[...]
    n = n_ref[...]
    lp = jnp.minimum(p, 0.0) - jnp.log1p(jnp.exp(-jnp.abs(p)))
    ln = jnp.minimum(-n, 0.0) - jnp.log1p(jnp.exp(-jnp.abs(n)))
    o_ref[0, 0] = -jnp.sum(lp + ln) * (1.0 / B)


_tc_loss = pl.pallas_call(
    _tc_body,
    out_shape=jax.ShapeDtypeStruct((1, 1), jnp.float32),
    out_specs=pl.BlockSpec(memory_space=pltpu.SMEM),
)


def kernel(target, context, negative_word_batch, input_emb, output_emb):
    neg_t = jnp.transpose(negative_word_batch)
    iemb = _tc_transpose(input_emb.T).reshape(VROWS, D)
    oemb = _tc_transpose(output_emb.T).reshape(VROWS, D)
    pdot, ndot = _sc_dots(
        _remap(target), _remap(context), _remap(neg_t), iemb, oemb
    )
    out = _tc_loss(pdot.reshape(128, 128), ndot.reshape(128, 128))
    return out.reshape(())


# speedup vs baseline: 1.7215x; 1.7215x over previous
import functools

import numpy as np

import jax
import jax.numpy as jnp
from jax import lax
from jax.experimental import pallas as pl
from jax.experimental.pallas import tpu as pltpu
from jax.experimental.pallas import tpu_sc as plsc

V = 1000000
D = 64
B = 16384
K = 20

NC = 2
NS = 16
NW = NC * NS
BPW = B // NW
CHUNK = 128
NCH = BPW // CHUNK
GRPS = CHUNK // 16

TBLK = 4096
NQ = (V + TBLK - 1) // TBLK
VROWS = 2 * TBLK * ((NQ + 1) // 2)


def _tr_body(x_ref, o_ref):
    q = pl.program_id(0)
    t = x_ref[...].T

    @pl.when(q % 2 == 0)
    def _():
        o_ref[:, 0:D] = t

    @pl.when(q % 2 == 1)
    def _():
        o_ref[:, D:2 * D] = t


_tc_transpose = pl.pallas_call(
    _tr_body,
    grid=(NQ,),
    in_specs=[pl.BlockSpec((D, TBLK), lambda q: (0, q))],
    out_specs=pl.BlockSpec((TBLK, 2 * D), lambda q: (q // 2, 0)),
    out_shape=jax.ShapeDtypeStruct((VROWS // 2, 2 * D), jnp.float32),
)


def _remap(r):
    q = r // TBLK
    return 2 * TBLK * (q >> 1) + 2 * (r % TBLK) + (q & 1)


def _dots16(buf, vbuf, scr, g):
    def row(j, _):
        r = g * 16 + j
        p = buf[r, pl.ds(0, 16)] * vbuf[r, pl.ds(0, 16)]
        p = p + buf[r, pl.ds(16, 16)] * vbuf[r, pl.ds(16, 16)]
        p = p + buf[r, pl.ds(32, 16)] * vbuf[r, pl.ds(32, 16)]
        p = p + buf[r, pl.ds(48, 16)] * vbuf[r, pl.ds(48, 16)]
        plsc.store_scatter(scr, [lax.iota(jnp.int32, 16) * 16 + j], p)
        return 0

    lax.fori_loop(0, 16, row, 0)

    def srow(i, a):
        return a + scr[pl.ds(i * 16, 16)]

    return lax.fori_loop(0, 16, srow, jnp.zeros((16,), jnp.float32))


_mesh = plsc.VectorSubcoreMesh(core_axis_name="c", subcore_axis_name="s")


@functools.partial(
    pl.kernel,
    mesh=_mesh,
    compiler_params=pltpu.CompilerParams(
        needs_layout_passes=False, use_tc_tiling_on_sc=False
    ),
    out_type=(
        jax.ShapeDtypeStruct((B,), jnp.float32),
        jax.ShapeDtypeStruct((B,), jnp.float32),
    ),
    scratch_types=[
        pltpu.VMEM((CHUNK,), jnp.int32),
        pltpu.VMEM((CHUNK,), jnp.int32),
        pltpu.VMEM((K, CHUNK), jnp.int32),
        pltpu.VMEM((CHUNK, D), jnp.float32),
        pltpu.VMEM((CHUNK, D), jnp.float32),
        pltpu.VMEM((CHUNK, D), jnp.float32),
        pltpu.VMEM((256,), jnp.float32),
        pltpu.VMEM((CHUNK,), jnp.float32),
        pltpu.VMEM((CHUNK,), jnp.float32),
        pltpu.SemaphoreType.DMA,
    ],
)
def _sc_dots(tgt, ctx, negt, iemb, oemb, pdot, ndot,
             tgti, ctxi, negi, vbuf, ubuf, nbuf, scr, pv, nv, sem):
    wid = lax.axis_index("s") * NC + lax.axis_index("c")
    for c in range(NCH):
        off = wid * BPW + c * CHUNK
        pltpu.sync_copy(tgt.at[pl.ds(off, CHUNK)], tgti)
        pltpu.sync_copy(ctx.at[pl.ds(off, CHUNK)], ctxi)
        pltpu.sync_copy(negt.at[:, pl.ds(off, CHUNK)], negi)
        pltpu.async_copy(iemb.at[tgti], vbuf, sem).wait()
        pltpu.async_copy(oemb.at[ctxi], ubuf, sem).wait()

        def pgrp(g, _):
            pv[pl.ds(g * 16, 16)] = _dots16(ubuf, vbuf, scr, g)
            return 0

        lax.fori_loop(0, GRPS, pgrp, 0)

        def zb(g, _):
            nv[pl.ds(g * 16, 16)] = jnp.zeros((16,), jnp.float32)
            return 0

        lax.fori_loop(0, GRPS, zb, 0)

        def kb(k, _):
            pltpu.async_copy(oemb.at[negi.at[k]], nbuf, sem).wait()

            def ngrp(g, _):
                nv[pl.ds(g * 16, 16)] = (
                    nv[pl.ds(g * 16, 16)] + _dots16(nbuf, vbuf, scr, g)
                )
                return 0

            lax.fori_loop(0, GRPS, ngrp, 0)
            return 0

        lax.fori_loop(0, K, kb, 0)

        pltpu.sync_copy(pv, pdot.at[pl.ds(off, CHUNK)])
        pltpu.sync_copy(nv, ndot.at[pl.ds(off, CHUNK)])


def _tc_body(p_ref, n_ref, o_ref):
    p = p_ref[...]
    n = n_ref[...]
    lp = jnp.minimum(p, 0.0) - jnp.log1p(jnp.exp(-jnp.abs(p)))
    ln = jnp.minimum(-n, 0.0) - jnp.log1p(jnp.exp(-jnp.abs(n)))
    o_ref[0, 0] = -jnp.sum(lp + ln) * (1.0 / B)


_tc_loss = pl.pallas_call(
    _tc_body,
    out_shape=jax.ShapeDtypeStruct((1, 1), jnp.float32),
    out_specs=pl.BlockSpec(memory_space=pltpu.SMEM),
)


def kernel(target, context, negative_word_batch, input_emb, output_emb):
    neg_t = jnp.transpose(negative_word_batch)
    iemb = _tc_transpose(input_emb.T).reshape(VROWS, D)
    oemb = _tc_transpose(output_emb.T).reshape(VROWS, D)
    pdot, ndot = _sc_dots(
        _remap(target), _remap(context), _remap(neg_t), iemb, oemb
    )
    out = _tc_loss(pdot.reshape(128, 128), ndot.reshape(128, 128))
    return out.reshape(())


# speedup vs baseline: 2.1325x; 1.2387x over previous
import functools

import numpy as np

import jax
import jax.numpy as jnp
from jax import lax
from jax.experimental import pallas as pl
from jax.experimental.pallas import tpu as pltpu
from jax.experimental.pallas import tpu_sc as plsc

V = 1000000
D = 64
B = 16384
K = 20

NC = 2
NS = 16
NW = NC * NS
BPW = B // NW
CHUNK = 128
NCH = BPW // CHUNK
GRPS = CHUNK // 16

TBLK = 4096
NQ = (V + TBLK - 1) // TBLK
VROWS = 2 * TBLK * ((NQ + 1) // 2)


def _tr_body(x_ref, o_ref):
    q = pl.program_id(0)
    t = x_ref[...].T

    @pl.when(q % 2 == 0)
    def _():
        o_ref[:, 0:D] = t

    @pl.when(q % 2 == 1)
    def _():
        o_ref[:, D:2 * D] = t


_tc_transpose = pl.pallas_call(
    _tr_body,
    grid=(NQ,),
    in_specs=[pl.BlockSpec((D, TBLK), lambda q: (0, q))],
    out_specs=pl.BlockSpec((TBLK, 2 * D), lambda q: (q // 2, 0)),
    out_shape=jax.ShapeDtypeStruct((VROWS // 2, 2 * D), jnp.float32),
)


def _remap(r):
    q = r // TBLK
    return 2 * TBLK * (q >> 1) + 2 * (r % TBLK) + (q & 1)


def _dots16(buf, vbuf, scr, g):
    def row(j, _):
        r = g * 16 + j
        p = buf[r, pl.ds(0, 16)] * vbuf[r, pl.ds(0, 16)]
        p = p + buf[r, pl.ds(16, 16)] * vbuf[r, pl.ds(16, 16)]
        p = p + buf[r, pl.ds(32, 16)] * vbuf[r, pl.ds(32, 16)]
        p = p + buf[r, pl.ds(48, 16)] * vbuf[r, pl.ds(48, 16)]
        plsc.store_scatter(scr, [lax.iota(jnp.int32, 16) * 16 + j], p)
        return 0

    lax.fori_loop(0, 16, row, 0)

    def srow(i, a):
        return a + scr[pl.ds(i * 16, 16)]

    return lax.fori_loop(0, 16, srow, jnp.zeros((16,), jnp.float32))


_mesh = plsc.VectorSubcoreMesh(core_axis_name="c", subcore_axis_name="s")
_SC_PARAMS = pltpu.CompilerParams(
    needs_layout_passes=False, use_tc_tiling_on_sc=False
)


@functools.partial(
    pl.kernel,
    mesh=_mesh,
    compiler_params=_SC_PARAMS,
    out_type=(
        jax.ShapeDtypeStruct((B, D), jnp.float32),
        jax.ShapeDtypeStruct((B, D), jnp.float32),
    ),
    scratch_types=[
        pltpu.VMEM((CHUNK,), jnp.int32),
        pltpu.VMEM((K, CHUNK), jnp.int32),
        pltpu.VMEM((CHUNK, D), jnp.float32),
        pltpu.VMEM((CHUNK, D), jnp.float32),
        pltpu.VMEM((CHUNK, D), jnp.float32),
        pltpu.VMEM((CHUNK,), jnp.int32),
        pltpu.VMEM_SHARED((NS * CHUNK, D), jnp.float32),
        pltpu.SemaphoreType.DMA,
        pltpu.SemaphoreType.DMA,
    ],
)
def _sc_stage(ctx, negt, oemb, uout, sout,
              ctxi, negi, ubuf, nbuf0, nbuf1, idxv, shared, sem, sem2):
    cid = lax.axis_index("c")
    sid = lax.axis_index("s")
    wid = sid * NC + cid

    def ib(g, _):
        idxv[pl.ds(g * 16, 16)] = (
            lax.iota(jnp.int32, 16) + (sid * CHUNK + g * 16)
        )
        return 0

    lax.fori_loop(0, GRPS, ib, 0)

    nbufs = (nbuf0, nbuf1)
    sems = (sem, sem2)
    for c in range(NCH):
        off = wid * BPW + c * CHUNK
        pltpu.sync_copy(ctx.at[pl.ds(off, CHUNK)], ctxi)
        pltpu.sync_copy(negt.at[:, pl.ds(off, CHUNK)], negi)
        pltpu.sync_copy(oemb.at[ctxi], ubuf)
        pltpu.sync_copy(ubuf, uout.at[pl.ds(off, CHUNK)])
        cp = pltpu.async_copy(oemb.at[negi.at[0]], nbufs[0], sems[0])
        for k in range(K):
            if k + 1 < K:
                nxt = pltpu.async_copy(
                    oemb.at[negi.at[k + 1]], nbufs[(k + 1) % 2],
                    sems[(k + 1) % 2],
                )
            cp.wait()
            pltpu.sync_copy(nbufs[k % 2], shared.at[idxv], add=(k > 0))
            if k + 1 < K:
                cp = nxt
        pltpu.sync_copy(
            shared.at[pl.ds(sid * CHUNK, CHUNK)], sout.at[pl.ds(off, CHUNK)]
        )


@functools.partial(
    pl.kernel,
    mesh=_mesh,
    compiler_params=_SC_PARAMS,
    out_type=(
        jax.ShapeDtypeStruct((B,), jnp.float32),
        jax.ShapeDtypeStruct((B,), jnp.float32),
    ),
    scratch_types=[
        pltpu.VMEM((CHUNK,), jnp.int32),
        pltpu.VMEM((CHUNK, D), jnp.float32),
        pltpu.VMEM((CHUNK, D), jnp.float32),
        pltpu.VMEM((CHUNK, D), jnp.float32),
        pltpu.VMEM((256,), jnp.float32),
        pltpu.VMEM((CHUNK,), jnp.float32),
        pltpu.VMEM((CHUNK,), jnp.float32),
        pltpu.SemaphoreType.DMA,
    ],
)
def _sc_dots2(tgt, uin, sin, iemb, pdot, ndot,
              tgti, vbuf, ub2, sb2, scr, pv, nv, sem):
    wid = lax.axis_index("s") * NC + lax.axis_index("c")
    for c in range(NCH):
        off = wid * BPW + c * CHUNK
        pltpu.sync_copy(tgt.at[pl.ds(off, CHUNK)], tgti)
        ucp = pltpu.async_copy(uin.at[pl.ds(off, CHUNK)], ub2, sem)
        scp = pltpu.async_copy(sin.at[pl.ds(off, CHUNK)], sb2, sem)
        pltpu.async_copy(iemb.at[tgti], vbuf, sem).wait()
        scp.wait()
        ucp.wait()

        def grp(g, _):
            pv[pl.ds(g * 16, 16)] = _dots16(ub2, vbuf, scr, g)
            nv[pl.ds(g * 16, 16)] = _dots16(sb2, vbuf, scr, g)
            return 0

        lax.fori_loop(0, GRPS, grp, 0)

        pltpu.sync_copy(pv, pdot.at[pl.ds(off, CHUNK)])
        pltpu.sync_copy(nv, ndot.at[pl.ds(off, CHUNK)])


def _tc_body(p_ref, n_ref, o_ref):
    p = p_ref[...]
    n = n_ref[...]
    lp = jnp.minimum(p, 0.0) - jnp.log1p(jnp.exp(-jnp.abs(p)))
    ln = jnp.minimum(-n, 0.0) - jnp.log1p(jnp.exp(-jnp.abs(n)))
    o_ref[0, 0] = -jnp.sum(lp + ln) * (1.0 / B)


_tc_loss = pl.pallas_call(
    _tc_body,
    out_shape=jax.ShapeDtypeStruct((1, 1), jnp.float32),
    out_specs=pl.BlockSpec(memory_space=pltpu.SMEM),
)


def kernel(target, context, negative_word_batch, input_emb, output_emb):
    neg_t = jnp.transpose(negative_word_batch)
    oemb = _tc_transpose(output_emb.T).reshape(VROWS, D)
    u_rows, s_rows = _sc_stage(_remap(context), _remap(neg_t), oemb)
    iemb = _tc_transpose(input_emb.T).reshape(VROWS, D)
    pdot, ndot = _sc_dots2(_remap(target), u_rows, s_rows, iemb)
    out = _tc_loss(pdot.reshape(128, 128), ndot.reshape(128, 128))
    return out.reshape(())


# speedup vs baseline: 2.5470x; 1.1944x over previous
import functools

import numpy as np

import jax
import jax.numpy as jnp
from jax import lax
from jax.experimental import pallas as pl
from jax.experimental.pallas import tpu as pltpu
from jax.experimental.pallas import tpu_sc as plsc

V = 1000000
D = 64
B = 16384
K = 20

NC = 2
NS = 16
NW = NC * NS
BPW = B // NW
CHUNK = 128
NCH = BPW // CHUNK
GRPS = CHUNK // 16

TBLK = 4096
HB = TBLK // 2
NQ = (V + TBLK - 1) // TBLK
VROWS = NQ * TBLK


def _tr_body(x_ref, o_ref):
    t = x_ref[...].T
    o_ref[...] = jnp.concatenate([t[:HB], t[HB:]], axis=1)


_tc_transpose = pl.pallas_call(
    _tr_body,
    grid=(NQ,),
    in_specs=[pl.BlockSpec((D, TBLK), lambda q: (0, q))],
    out_specs=pl.BlockSpec((HB, 2 * D), lambda q: (q, 0)),
    out_shape=jax.ShapeDtypeStruct((VROWS // 2, 2 * D), jnp.float32),
)


def _remap(r):
    p = r % TBLK
    return (r // TBLK) * TBLK + 2 * (p % HB) + (p // HB)


def _dots16(buf, vbuf, scr, g):
    def row(j, _):
        r = g * 16 + j
        p = buf[r, pl.ds(0, 16)] * vbuf[r, pl.ds(0, 16)]
        p = p + buf[r, pl.ds(16, 16)] * vbuf[r, pl.ds(16, 16)]
        p = p + buf[r, pl.ds(32, 16)] * vbuf[r, pl.ds(32, 16)]
        p = p + buf[r, pl.ds(48, 16)] * vbuf[r, pl.ds(48, 16)]
        plsc.store_scatter(scr, [lax.iota(jnp.int32, 16) * 16 + j], p)
        return 0

    lax.fori_loop(0, 16, row, 0)

    def srow(i, a):
        return a + scr[pl.ds(i * 16, 16)]

    return lax.fori_loop(0, 16, srow, jnp.zeros((16,), jnp.float32))


_mesh = plsc.VectorSubcoreMesh(core_axis_name="c", subcore_axis_name="s")
_SC_PARAMS = pltpu.CompilerParams(
    needs_layout_passes=False, use_tc_tiling_on_sc=False
)


@functools.partial(
    pl.kernel,
    mesh=_mesh,
    compiler_params=_SC_PARAMS,
    out_type=(
        jax.ShapeDtypeStruct((B, D), jnp.float32),
        jax.ShapeDtypeStruct((B, D), jnp.float32),
    ),
    scratch_types=[
        pltpu.VMEM((CHUNK,), jnp.int32),
        pltpu.VMEM((K, CHUNK), jnp.int32),
        pltpu.VMEM((CHUNK, D), jnp.float32),
        pltpu.VMEM((CHUNK, D), jnp.float32),
        pltpu.VMEM((CHUNK, D), jnp.float32),
        pltpu.VMEM((CHUNK,), jnp.int32),
        pltpu.VMEM_SHARED((NS * CHUNK, D), jnp.float32),
        pltpu.SemaphoreType.DMA,
        pltpu.SemaphoreType.DMA,
    ],
)
def _sc_stage(ctx, negt, oemb, uout, sout,
              ctxi, negi, ubuf, nbuf0, nbuf1, idxv, shared, sem, sem2):
    cid = lax.axis_index("c")
    sid = lax.axis_index("s")
    wid = sid * NC + cid

    def ib(g, _):
        idxv[pl.ds(g * 16, 16)] = (
            lax.iota(jnp.int32, 16) + (sid * CHUNK + g * 16)
        )
        return 0

    lax.fori_loop(0, GRPS, ib, 0)

    nbufs = (nbuf0, nbuf1)
    sems = (sem, sem2)
    for c in range(NCH):
        off = wid * BPW + c * CHUNK
        pltpu.sync_copy(ctx.at[pl.ds(off, CHUNK)], ctxi)
        pltpu.sync_copy(negt.at[:, pl.ds(off, CHUNK)], negi)
        pltpu.sync_copy(oemb.at[ctxi], ubuf)
        pltpu.sync_copy(ubuf, uout.at[pl.ds(off, CHUNK)])
        cp = pltpu.async_copy(oemb.at[negi.at[0]], nbufs[0], sems[0])
        for k in range(K):
            if k + 1 < K:
                nxt = pltpu.async_copy(
                    oemb.at[negi.at[k + 1]], nbufs[(k + 1) % 2],
                    sems[(k + 1) % 2],
                )
            cp.wait()
            pltpu.sync_copy(nbufs[k % 2], shared.at[idxv], add=(k > 0))
            if k + 1 < K:
                cp = nxt
        pltpu.sync_copy(
            shared.at[pl.ds(sid * CHUNK, CHUNK)], sout.at[pl.ds(off, CHUNK)]
        )


@functools.partial(
    pl.kernel,
    mesh=_mesh,
    compiler_params=_SC_PARAMS,
    out_type=(
        jax.ShapeDtypeStruct((B,), jnp.float32),
        jax.ShapeDtypeStruct((B,), jnp.float32),
    ),
    scratch_types=[
        pltpu.VMEM((CHUNK,), jnp.int32),
        pltpu.VMEM((CHUNK, D), jnp.float32),
        pltpu.VMEM((CHUNK, D), jnp.float32),
        pltpu.VMEM((CHUNK, D), jnp.float32),
        pltpu.VMEM((256,), jnp.float32),
        pltpu.VMEM((CHUNK,), jnp.float32),
        pltpu.VMEM((CHUNK,), jnp.float32),
        pltpu.SemaphoreType.DMA,
    ],
)
def _sc_dots2(tgt, uin, sin, iemb, pdot, ndot,
              tgti, vbuf, ub2, sb2, scr, pv, nv, sem):
    wid = lax.axis_index("s") * NC + lax.axis_index("c")
    for c in range(NCH):
        off = wid * BPW + c * CHUNK
        pltpu.sync_copy(tgt.at[pl.ds(off, CHUNK)], tgti)
        ucp = pltpu.async_copy(uin.at[pl.ds(off, CHUNK)], ub2, sem)
        scp = pltpu.async_copy(sin.at[pl.ds(off, CHUNK)], sb2, sem)
        pltpu.async_copy(iemb.at[tgti], vbuf, sem).wait()
        scp.wait()
        ucp.wait()

        def grp(g, _):
            pv[pl.ds(g * 16, 16)] = _dots16(ub2, vbuf, scr, g)
            nv[pl.ds(g * 16, 16)] = _dots16(sb2, vbuf, scr, g)
            return 0

        lax.fori_loop(0, GRPS, grp, 0)

        pltpu.sync_copy(pv, pdot.at[pl.ds(off, CHUNK)])
        pltpu.sync_copy(nv, ndot.at[pl.ds(off, CHUNK)])


def _tc_body(p_ref, n_ref, o_ref):
    p = p_ref[...]
    n = n_ref[...]
    lp = jnp.minimum(p, 0.0) - jnp.log1p(jnp.exp(-jnp.abs(p)))
    ln = jnp.minimum(-n, 0.0) - jnp.log1p(jnp.exp(-jnp.abs(n)))
    o_ref[0, 0] = -jnp.sum(lp + ln) * (1.0 / B)


_tc_loss = pl.pallas_call(
    _tc_body,
    out_shape=jax.ShapeDtypeStruct((1, 1), jnp.float32),
    out_specs=pl.BlockSpec(memory_space=pltpu.SMEM),
)


def kernel(target, context, negative_word_batch, input_emb, output_emb):
    neg_t = jnp.transpose(negative_word_batch)
    oemb = _tc_transpose(output_emb.T).reshape(VROWS, D)
    u_rows, s_rows = _sc_stage(_remap(context), _remap(neg_t), oemb)
    iemb = _tc_transpose(input_emb.T).reshape(VROWS, D)
    pdot, ndot = _sc_dots2(_remap(target), u_rows, s_rows, iemb)
    out = _tc_loss(pdot.reshape(128, 128), ndot.reshape(128, 128))
    return out.reshape(())


# speedup vs baseline: 3.1574x; 1.2396x over previous
import functools

import numpy as np

import jax
import jax.numpy as jnp
from jax import lax
from jax.experimental import pallas as pl
from jax.experimental.pallas import tpu as pltpu
from jax.experimental.pallas import tpu_sc as plsc

V = 1000000
D = 64
B = 16384
K = 20

NC = 2
NS = 16
NW = NC * NS
BPW = B // NW
CHUNK = 128
NCH = BPW // CHUNK
GRPS = CHUNK // 16

TBLK = 8192
HB = TBLK // 2
NQ = (V + TBLK - 1) // TBLK
VROWS = NQ * TBLK


def _tr_body(x_ref, o_ref):
    t = x_ref[...].T
    o_ref[...] = jnp.concatenate([t[:HB], t[HB:]], axis=1)


_tc_transpose = pl.pallas_call(
    _tr_body,
    grid=(NQ,),
    in_specs=[pl.BlockSpec((D, TBLK), lambda q: (0, q))],
    out_specs=pl.BlockSpec((HB, 2 * D), lambda q: (q, 0)),
    out_shape=jax.ShapeDtypeStruct((VROWS // 2, 2 * D), jnp.float32),
)


def _remap(r):
    p = r % TBLK
    return (r // TBLK) * TBLK + 2 * (p % HB) + (p // HB)


def _dots16(buf, vbuf, scr, g):
    def row(j, _):
        r = g * 16 + j
        p = buf[r, pl.ds(0, 16)] * vbuf[r, pl.ds(0, 16)]
        p = p + buf[r, pl.ds(16, 16)] * vbuf[r, pl.ds(16, 16)]
        p = p + buf[r, pl.ds(32, 16)] * vbuf[r, pl.ds(32, 16)]
        p = p + buf[r, pl.ds(48, 16)] * vbuf[r, pl.ds(48, 16)]
        plsc.store_scatter(scr, [lax.iota(jnp.int32, 16) * 16 + j], p)
        return 0

    lax.fori_loop(0, 16, row, 0)

    def srow(i, a):
        return a + scr[pl.ds(i * 16, 16)]

    return lax.fori_loop(0, 16, srow, jnp.zeros((16,), jnp.float32))


_mesh = plsc.VectorSubcoreMesh(core_axis_name="c", subcore_axis_name="s")
_SC_PARAMS = pltpu.CompilerParams(
    needs_layout_passes=False, use_tc_tiling_on_sc=False
)


@functools.partial(
    pl.kernel,
    mesh=_mesh,
    compiler_params=_SC_PARAMS,
    out_type=(
        jax.ShapeDtypeStruct((B, D), jnp.float32),
        jax.ShapeDtypeStruct((B, D), jnp.float32),
    ),
    scratch_types=[
        pltpu.VMEM((CHUNK,), jnp.int32),
        pltpu.VMEM((K, CHUNK), jnp.int32),
        pltpu.VMEM((CHUNK, D), jnp.float32),
        pltpu.VMEM((CHUNK, D), jnp.float32),
        pltpu.VMEM((CHUNK, D), jnp.float32),
        pltpu.VMEM((CHUNK,), jnp.int32),
        pltpu.VMEM_SHARED((NS * CHUNK, D), jnp.float32),
        pltpu.SemaphoreType.DMA,
        pltpu.SemaphoreType.DMA,
    ],
)
def _sc_stage(ctx, negt, oemb, uout, sout,
              ctxi, negi, ubuf, nbuf0, nbuf1, idxv, shared, sem, sem2):
    cid = lax.axis_index("c")
    sid = lax.axis_index("s")
    wid = sid * NC + cid

    def ib(g, _):
        idxv[pl.ds(g * 16, 16)] = (
            lax.iota(jnp.int32, 16) + (sid * CHUNK + g * 16)
        )
        return 0

    lax.fori_loop(0, GRPS, ib, 0)

    nbufs = (nbuf0, nbuf1)
    sems = (sem, sem2)
    for c in range(NCH):
        off = wid * BPW + c * CHUNK
        pltpu.sync_copy(ctx.at[pl.ds(off, CHUNK)], ctxi)
        pltpu.sync_copy(negt.at[:, pl.ds(off, CHUNK)], negi)
        pltpu.sync_copy(oemb.at[ctxi], ubuf)
        pltpu.sync_copy(ubuf, uout.at[pl.ds(off, CHUNK)])
        cp = pltpu.async_copy(oemb.at[negi.at[0]], nbufs[0], sems[0])
        for k in range(K):
            if k + 1 < K:
                nxt = pltpu.async_copy(
                    oemb.at[negi.at[k + 1]], nbufs[(k + 1) % 2],
                    sems[(k + 1) % 2],
                )
            cp.wait()
            pltpu.sync_copy(nbufs[k % 2], shared.at[idxv], add=(k > 0))
            if k + 1 < K:
                cp = nxt
        pltpu.sync_copy(
            shared.at[pl.ds(sid * CHUNK, CHUNK)], sout.at[pl.ds(off, CHUNK)]
        )


@functools.partial(
    pl.kernel,
    mesh=_mesh,
    compiler_params=_SC_PARAMS,
    out_type=(
        jax.ShapeDtypeStruct((B,), jnp.float32),
        jax.ShapeDtypeStruct((B,), jnp.float32),
    ),
    scratch_types=[
        pltpu.VMEM((CHUNK,), jnp.int32),
        pltpu.VMEM((CHUNK, D), jnp.float32),
        pltpu.VMEM((CHUNK, D), jnp.float32),
        pltpu.VMEM((CHUNK, D), jnp.float32),
        pltpu.VMEM((256,), jnp.float32),
        pltpu.VMEM((CHUNK,), jnp.float32),
        pltpu.VMEM((CHUNK,), jnp.float32),
        pltpu.SemaphoreType.DMA,
    ],
)
def _sc_dots2(tgt, uin, sin, iemb, pdot, ndot,
              tgti, vbuf, ub2, sb2, scr, pv, nv, sem):
    wid = lax.axis_index("s") * NC + lax.axis_index("c")
    for c in range(NCH):
        off = wid * BPW + c * CHUNK
        pltpu.sync_copy(tgt.at[pl.ds(off, CHUNK)], tgti)
        ucp = pltpu.async_copy(uin.at[pl.ds(off, CHUNK)], ub2, sem)
        scp = pltpu.async_copy(sin.at[pl.ds(off, CHUNK)], sb2, sem)
        pltpu.async_copy(iemb.at[tgti], vbuf, sem).wait()
        scp.wait()
        ucp.wait()

        def grp(g, _):
            pv[pl.ds(g * 16, 16)] = _dots16(ub2, vbuf, scr, g)
            nv[pl.ds(g * 16, 16)] = _dots16(sb2, vbuf, scr, g)
            return 0

        lax.fori_loop(0, GRPS, grp, 0)

        pltpu.sync_copy(pv, pdot.at[pl.ds(off, CHUNK)])
        pltpu.sync_copy(nv, ndot.at[pl.ds(off, CHUNK)])


def _tc_body(p_ref, n_ref, o_ref):
    p = p_ref[...]
    n = n_ref[...]
    lp = jnp.minimum(p, 0.0) - jnp.log1p(jnp.exp(-jnp.abs(p)))
    ln = jnp.minimum(-n, 0.0) - jnp.log1p(jnp.exp(-jnp.abs(n)))
    o_ref[0, 0] = -jnp.sum(lp + ln) * (1.0 / B)


_tc_loss = pl.pallas_call(
    _tc_body,
    out_shape=jax.ShapeDtypeStruct((1, 1), jnp.float32),
    out_specs=pl.BlockSpec(memory_space=pltpu.SMEM),
)


def kernel(target, context, negative_word_batch, input_emb, output_emb):
    neg_t = jnp.transpose(negative_word_batch)
    oemb = _tc_transpose(output_emb.T).reshape(VROWS, D)
    u_rows, s_rows = _sc_stage(_remap(context), _remap(neg_t), oemb)
    iemb = _tc_transpose(input_emb.T).reshape(VROWS, D)
    pdot, ndot = _sc_dots2(_remap(target), u_rows, s_rows, iemb)
    out = _tc_loss(pdot.reshape(128, 128), ndot.reshape(128, 128))
    return out.reshape(())


# speedup vs baseline: 3.5307x; 1.1182x over previous
import functools

import numpy as np

import jax
import jax.numpy as jnp
from jax import lax
from jax.experimental import pallas as pl
from jax.experimental.pallas import tpu as pltpu
from jax.experimental.pallas import tpu_sc as plsc

V = 1000000
D = 64
B = 16384
K = 20

NC = 2
NS = 16
NW = NC * NS
BPW = B // NW
CHUNK = 128
NCH = BPW // CHUNK
GRPS = CHUNK // 16

TBLK = 16384
HB = TBLK // 2
NQ = (V + TBLK - 1) // TBLK
VROWS = NQ * TBLK


def _tr_body(x_ref, o_ref):
    t = x_ref[...].T
    o_ref[...] = jnp.concatenate([t[:HB], t[HB:]], axis=1)


_tc_transpose = pl.pallas_call(
    _tr_body,
    grid=(NQ,),
    in_specs=[pl.BlockSpec((D, TBLK), lambda q: (0, q))],
    out_specs=pl.BlockSpec((HB, 2 * D), lambda q: (q, 0)),
    out_shape=jax.ShapeDtypeStruct((VROWS // 2, 2 * D), jnp.float32),
)


def _remap(r):
    p = r % TBLK
    return (r // TBLK) * TBLK + 2 * (p % HB) + (p // HB)


def _dots16(buf, vbuf, scr, g):
    def row(j, _):
        r = g * 16 + j
        p = buf[r, pl.ds(0, 16)] * vbuf[r, pl.ds(0, 16)]
        p = p + buf[r, pl.ds(16, 16)] * vbuf[r, pl.ds(16, 16)]
        p = p + buf[r, pl.ds(32, 16)] * vbuf[r, pl.ds(32, 16)]
        p = p + buf[r, pl.ds(48, 16)] * vbuf[r, pl.ds(48, 16)]
        plsc.store_scatter(scr, [lax.iota(jnp.int32, 16) * 16 + j], p)
        return 0

    lax.fori_loop(0, 16, row, 0)

    def srow(i, a):
        return a + scr[pl.ds(i * 16, 16)]

    return lax.fori_loop(0, 16, srow, jnp.zeros((16,), jnp.float32))


_mesh = plsc.VectorSubcoreMesh(core_axis_name="c", subcore_axis_name="s")
_SC_PARAMS = pltpu.CompilerParams(
    needs_layout_passes=False, use_tc_tiling_on_sc=False
)


@functools.partial(
    pl.kernel,
    mesh=_mesh,
    compiler_params=_SC_PARAMS,
    out_type=(
        jax.ShapeDtypeStruct((B, D), jnp.float32),
        jax.ShapeDtypeStruct((B, D), jnp.float32),
    ),
    scratch_types=[
        pltpu.VMEM((CHUNK,), jnp.int32),
        pltpu.VMEM((K, CHUNK), jnp.int32),
        pltpu.VMEM((CHUNK, D), jnp.float32),
        pltpu.VMEM((CHUNK, D), jnp.float32),
        pltpu.VMEM((CHUNK, D), jnp.float32),
        pltpu.VMEM((CHUNK,), jnp.int32),
        pltpu.VMEM_SHARED((NS * CHUNK, D), jnp.float32),
        pltpu.SemaphoreType.DMA,
        pltpu.SemaphoreType.DMA,
    ],
)
def _sc_stage(ctx, negt, oemb, uout, sout,
              ctxi, negi, ubuf, nbuf0, nbuf1, idxv, shared, sem, sem2):
    cid = lax.axis_index("c")
    sid = lax.axis_index("s")
    wid = sid * NC + cid

    def ib(g, _):
        idxv[pl.ds(g * 16, 16)] = (
            lax.iota(jnp.int32, 16) + (sid * CHUNK + g * 16)
        )
        return 0

    lax.fori_loop(0, GRPS, ib, 0)

    nbufs = (nbuf0, nbuf1)
    sems = (sem, sem2)
    for c in range(NCH):
        off = wid * BPW + c * CHUNK
        pltpu.sync_copy(ctx.at[pl.ds(off, CHUNK)], ctxi)
        pltpu.sync_copy(negt.at[:, pl.ds(off, CHUNK)], negi)
        pltpu.sync_copy(oemb.at[ctxi], ubuf)
        pltpu.sync_copy(ubuf, uout.at[pl.ds(off, CHUNK)])
        cp = pltpu.async_copy(oemb.at[negi.at[0]], nbufs[0], sems[0])
        for k in range(K):
            if k + 1 < K:
                nxt = pltpu.async_copy(
                    oemb.at[negi.at[k + 1]], nbufs[(k + 1) % 2],
                    sems[(k + 1) % 2],
                )
            cp.wait()
            pltpu.sync_copy(nbufs[k % 2], shared.at[idxv], add=(k > 0))
            if k + 1 < K:
                cp = nxt
        pltpu.sync_copy(
            shared.at[pl.ds(sid * CHUNK, CHUNK)], sout.at[pl.ds(off, CHUNK)]
        )


@functools.partial(
    pl.kernel,
    mesh=_mesh,
    compiler_params=_SC_PARAMS,
    out_type=(
        jax.ShapeDtypeStruct((B,), jnp.float32),
        jax.ShapeDtypeStruct((B,), jnp.float32),
    ),
    scratch_types=[
        pltpu.VMEM((CHUNK,), jnp.int32),
        pltpu.VMEM((CHUNK, D), jnp.float32),
        pltpu.VMEM((CHUNK, D), jnp.float32),
        pltpu.VMEM((CHUNK, D), jnp.float32),
        pltpu.VMEM((256,), jnp.float32),
        pltpu.VMEM((CHUNK,), jnp.float32),
        pltpu.VMEM((CHUNK,), jnp.float32),
        pltpu.SemaphoreType.DMA,
    ],
)
def _sc_dots2(tgt, uin, sin, iemb, pdot, ndot,
              tgti, vbuf, ub2, sb2, scr, pv, nv, sem):
    wid = lax.axis_index("s") * NC + lax.axis_index("c")
    for c in range(NCH):
        off = wid * BPW + c * CHUNK
        pltpu.sync_copy(tgt.at[pl.ds(off, CHUNK)], tgti)
        ucp = pltpu.async_copy(uin.at[pl.ds(off, CHUNK)], ub2, sem)
        scp = pltpu.async_copy(sin.at[pl.ds(off, CHUNK)], sb2, sem)
        pltpu.async_copy(iemb.at[tgti], vbuf, sem).wait()
        scp.wait()
        ucp.wait()

        def grp(g, _):
            pv[pl.ds(g * 16, 16)] = _dots16(ub2, vbuf, scr, g)
            nv[pl.ds(g * 16, 16)] = _dots16(sb2, vbuf, scr, g)
            return 0

        lax.fori_loop(0, GRPS, grp, 0)

        pltpu.sync_copy(pv, pdot.at[pl.ds(off, CHUNK)])
        pltpu.sync_copy(nv, ndot.at[pl.ds(off, CHUNK)])


def _tc_body(p_ref, n_ref, o_ref):
    p = p_ref[...]
    n = n_ref[...]
    lp = jnp.minimum(p, 0.0) - jnp.log1p(jnp.exp(-jnp.abs(p)))
    ln = jnp.minimum(-n, 0.0) - jnp.log1p(jnp.exp(-jnp.abs(n)))
    o_ref[0, 0] = -jnp.sum(lp + ln) * (1.0 / B)


_tc_loss = pl.pallas_call(
    _tc_body,
    out_shape=jax.ShapeDtypeStruct((1, 1), jnp.float32),
    out_specs=pl.BlockSpec(memory_space=pltpu.SMEM),
)


def kernel(target, context, negative_word_batch, input_emb, output_emb):
    neg_t = jnp.transpose(negative_word_batch)
    oemb = _tc_transpose(output_emb.T).reshape(VROWS, D)
    u_rows, s_rows = _sc_stage(_remap(context), _remap(neg_t), oemb)
    iemb = _tc_transpose(input_emb.T).reshape(VROWS, D)
    pdot, ndot = _sc_dots2(_remap(target), u_rows, s_rows, iemb)
    out = _tc_loss(pdot.reshape(128, 128), ndot.reshape(128, 128))
    return out.reshape(())


# speedup vs baseline: 3.7346x; 1.0577x over previous
import functools

import numpy as np

import jax
import jax.numpy as jnp
from jax import lax
from jax.experimental import pallas as pl
from jax.experimental.pallas import tpu as pltpu
from jax.experimental.pallas import tpu_sc as plsc

V = 1000000
D = 64
B = 16384
K = 20

NC = 2
NS = 16
NW = NC * NS
BPW = B // NW
CHUNK = 128
NCH = BPW // CHUNK
GRPS = CHUNK // 16

TBLK = 32768
HB = TBLK // 2
NQ = (V + TBLK - 1) // TBLK
VROWS = NQ * TBLK


def _tr_body(x_ref, o_ref):
    t = x_ref[...].T
    o_ref[...] = jnp.concatenate([t[:HB], t[HB:]], axis=1)


_tc_transpose = pl.pallas_call(
    _tr_body,
    grid=(NQ,),
    in_specs=[pl.BlockSpec((D, TBLK), lambda q: (0, q))],
    out_specs=pl.BlockSpec((HB, 2 * D), lambda q: (q, 0)),
    out_shape=jax.ShapeDtypeStruct((VROWS // 2, 2 * D), jnp.float32),
)


def _remap(r):
    p = r % TBLK
    return (r // TBLK) * TBLK + 2 * (p % HB) + (p // HB)


def _dots16(buf, vbuf, scr, g):
    def row(j, _):
        r = g * 16 + j
        p = buf[r, pl.ds(0, 16)] * vbuf[r, pl.ds(0, 16)]
        p = p + buf[r, pl.ds(16, 16)] * vbuf[r, pl.ds(16, 16)]
        p = p + buf[r, pl.ds(32, 16)] * vbuf[r, pl.ds(32, 16)]
        p = p + buf[r, pl.ds(48, 16)] * vbuf[r, pl.ds(48, 16)]
        plsc.store_scatter(scr, [lax.iota(jnp.int32, 16) * 16 + j], p)
        return 0

    lax.fori_loop(0, 16, row, 0)

    def srow(i, a):
        return a + scr[pl.ds(i * 16, 16)]

    return lax.fori_loop(0, 16, srow, jnp.zeros((16,), jnp.float32))


_mesh = plsc.VectorSubcoreMesh(core_axis_name="c", subcore_axis_name="s")
_SC_PARAMS = pltpu.CompilerParams(
    needs_layout_passes=False, use_tc_tiling_on_sc=False
)


@functools.partial(
    pl.kernel,
    mesh=_mesh,
    compiler_params=_SC_PARAMS,
    out_type=(
        jax.ShapeDtypeStruct((B, D), jnp.float32),
        jax.ShapeDtypeStruct((B, D), jnp.float32),
    ),
    scratch_types=[
        pltpu.VMEM((CHUNK,), jnp.int32),
        pltpu.VMEM((K, CHUNK), jnp.int32),
        pltpu.VMEM((CHUNK, D), jnp.float32),
        pltpu.VMEM((CHUNK, D), jnp.float32),
        pltpu.VMEM((CHUNK, D), jnp.float32),
        pltpu.VMEM((CHUNK,), jnp.int32),
        pltpu.VMEM_SHARED((NS * CHUNK, D), jnp.float32),
        pltpu.SemaphoreType.DMA,
        pltpu.SemaphoreType.DMA,
    ],
)
def _sc_stage(ctx, negt, oemb, uout, sout,
              ctxi, negi, ubuf, nbuf0, nbuf1, idxv, shared, sem, sem2):
    cid = lax.axis_index("c")
    sid = lax.axis_index("s")
    wid = sid * NC + cid

    def ib(g, _):
        idxv[pl.ds(g * 16, 16)] = (
            lax.iota(jnp.int32, 16) + (sid * CHUNK + g * 16)
        )
        return 0

    lax.fori_loop(0, GRPS, ib, 0)

    nbufs = (nbuf0, nbuf1)
    sems = (sem, sem2)
    for c in range(NCH):
        off = wid * BPW + c * CHUNK
        pltpu.sync_copy(ctx.at[pl.ds(off, CHUNK)], ctxi)
        pltpu.sync_copy(negt.at[:, pl.ds(off, CHUNK)], negi)
        pltpu.sync_copy(oemb.at[ctxi], ubuf)
        pltpu.sync_copy(ubuf, uout.at[pl.ds(off, CHUNK)])
        cp = pltpu.async_copy(oemb.at[negi.at[0]], nbufs[0], sems[0])
        for k in range(K):
            if k + 1 < K:
                nxt = pltpu.async_copy(
                    oemb.at[negi.at[k + 1]], nbufs[(k + 1) % 2],
                    sems[(k + 1) % 2],
                )
            cp.wait()
            pltpu.sync_copy(nbufs[k % 2], shared.at[idxv], add=(k > 0))
            if k + 1 < K:
                cp = nxt
        pltpu.sync_copy(
            shared.at[pl.ds(sid * CHUNK, CHUNK)], sout.at[pl.ds(off, CHUNK)]
        )


@functools.partial(
    pl.kernel,
    mesh=_mesh,
    compiler_params=_SC_PARAMS,
    out_type=(
        jax.ShapeDtypeStruct((B,), jnp.float32),
        jax.ShapeDtypeStruct((B,), jnp.float32),
    ),
    scratch_types=[
        pltpu.VMEM((CHUNK,), jnp.int32),
        pltpu.VMEM((CHUNK, D), jnp.float32),
        pltpu.VMEM((CHUNK, D), jnp.float32),
        pltpu.VMEM((CHUNK, D), jnp.float32),
        pltpu.VMEM((256,), jnp.float32),
        pltpu.VMEM((CHUNK,), jnp.float32),
        pltpu.VMEM((CHUNK,), jnp.float32),
        pltpu.SemaphoreType.DMA,
    ],
)
def _sc_dots2(tgt, uin, sin, iemb, pdot, ndot,
              tgti, vbuf, ub2, sb2, scr, pv, nv, sem):
    wid = lax.axis_index("s") * NC + lax.axis_index("c")
    for c in range(NCH):
        off = wid * BPW + c * CHUNK
        pltpu.sync_copy(tgt.at[pl.ds(off, CHUNK)], tgti)
        ucp = pltpu.async_copy(uin.at[pl.ds(off, CHUNK)], ub2, sem)
        scp = pltpu.async_copy(sin.at[pl.ds(off, CHUNK)], sb2, sem)
        pltpu.async_copy(iemb.at[tgti], vbuf, sem).wait()
        scp.wait()
        ucp.wait()

        def grp(g, _):
            pv[pl.ds(g * 16, 16)] = _dots16(ub2, vbuf, scr, g)
            nv[pl.ds(g * 16, 16)] = _dots16(sb2, vbuf, scr, g)
            return 0

        lax.fori_loop(0, GRPS, grp, 0)

        pltpu.sync_copy(pv, pdot.at[pl.ds(off, CHUNK)])
        pltpu.sync_copy(nv, ndot.at[pl.ds(off, CHUNK)])


def _tc_body(p_ref, n_ref, o_ref):
    p = p_ref[...]
    n = n_ref[...]
    lp = jnp.minimum(p, 0.0) - jnp.log1p(jnp.exp(-jnp.abs(p)))
    ln = jnp.minimum(-n, 0.0) - jnp.log1p(jnp.exp(-jnp.abs(n)))
    o_ref[0, 0] = -jnp.sum(lp + ln) * (1.0 / B)


_tc_loss = pl.pallas_call(
    _tc_body,
    out_shape=jax.ShapeDtypeStruct((1, 1), jnp.float32),
    out_specs=pl.BlockSpec(memory_space=pltpu.SMEM),
)


def kernel(target, context, negative_word_batch, input_emb, output_emb):
    neg_t = jnp.transpose(negative_word_batch)
    oemb = _tc_transpose(output_emb.T).reshape(VROWS, D)
    u_rows, s_rows = _sc_stage(_remap(context), _remap(neg_t), oemb)
    iemb = _tc_transpose(input_emb.T).reshape(VROWS, D)
    pdot, ndot = _sc_dots2(_remap(target), u_rows, s_rows, iemb)
    out = _tc_loss(pdot.reshape(128, 128), ndot.reshape(128, 128))
    return out.reshape(())
